# Initial kernel scaffold; baseline (speedup 1.0000x reference)
import jax
import jax.numpy as jnp
from jax.experimental import pallas as pl


def kernel(x, edge_index, W1, b1, W2, b2, Wc1, bc1, Wc3, bc3):
    raise NotImplementedError("write your pallas kernel here")


# speedup vs baseline: 6.2003x; 6.2003x over previous
import functools

import jax
import jax.numpy as jnp
from jax import lax
from jax.experimental import pallas as pl
from jax.experimental.pallas import tpu as pltpu
from jax.experimental.pallas import tpu_sc as plsc

N = 10000
NP = 10240
E = 320000
D_IN = 128
H = 256

NC = 2
NS = 16
CHUNK = 125
EROWS = E // CHUNK
ROWS_PER_W = EROWS // (NC * NS)
ROWS_PER_S = EROWS // NS
NPS = NP // NS
ZR = 64

_mesh = plsc.VectorSubcoreMesh(core_axis_name="c", subcore_axis_name="s")

BN = 1024


def _fill_const(buf, nrows, width, val):
    v = jnp.full((16,), val, jnp.float32)

    def body(i, carry):
        for k in range(width // 16):
            buf[i, pl.ds(k * 16, 16)] = v
        return carry

    lax.fori_loop(0, nrows, body, 0)


@functools.partial(
    pl.kernel,
    out_type=jax.ShapeDtypeStruct((NC, NP, 16), jnp.float32),
    mesh=_mesh,
    scratch_types=[
        pltpu.VMEM((ROWS_PER_S, CHUNK), jnp.int32),
        pltpu.VMEM((CHUNK, 16), jnp.float32),
        pltpu.VMEM((ZR, 16), jnp.float32),
        pltpu.VMEM_SHARED((NP, 16), jnp.float32),
    ],
)
def _deg_kernel(src_hbm, dst_hbm, out_hbm, idx_v, ones_v, zbuf, deg_sh):
    c = lax.axis_index("c")
    s = lax.axis_index("s")
    _fill_const(ones_v, CHUNK, 16, 1.0)
    _fill_const(zbuf, ZR, 16, 0.0)
    for k in range(NPS // ZR):
        pltpu.sync_copy(zbuf, deg_sh.at[pl.ds(s * NPS + k * ZR, ZR)])
    plsc.subcore_barrier()

    @pl.when(c == 0)
    def _():
        pltpu.sync_copy(src_hbm.at[pl.ds(s * ROWS_PER_S, ROWS_PER_S)], idx_v)

    @pl.when(c == 1)
    def _():
        pltpu.sync_copy(dst_hbm.at[pl.ds(s * ROWS_PER_S, ROWS_PER_S)], idx_v)

    def chunk(j, carry):
        pltpu.sync_copy(ones_v, deg_sh.at[idx_v.at[j]], add=True)
        return carry

    lax.fori_loop(0, ROWS_PER_S, chunk, 0)
    plsc.subcore_barrier()
    pltpu.sync_copy(deg_sh.at[pl.ds(s * NPS, NPS)],
                    out_hbm.at[c, pl.ds(s * NPS, NPS)])


@functools.partial(
    pl.kernel,
    out_type=jax.ShapeDtypeStruct((NC, NP, D_IN), jnp.float32),
    mesh=_mesh,
    scratch_types=[
        pltpu.VMEM((ROWS_PER_W, CHUNK), jnp.int32),
        pltpu.VMEM((ROWS_PER_W, CHUNK), jnp.int32),
        pltpu.VMEM((CHUNK, D_IN), jnp.float32),
        pltpu.VMEM((ZR, D_IN), jnp.float32),
        pltpu.VMEM_SHARED((NP, D_IN), jnp.float32),
        pltpu.SemaphoreType.DMA,
    ],
)
def _agg1_kernel(table_hbm, src_hbm, dst_hbm, out_hbm,
                 src_v, dst_v, msg_v, zbuf, agg_sh, sem):
    c = lax.axis_index("c")
    s = lax.axis_index("s")
    _fill_const(zbuf, ZR, D_IN, 0.0)
    for k in range(NPS // ZR):
        pltpu.sync_copy(zbuf, agg_sh.at[pl.ds(s * NPS + k * ZR, ZR)])
    plsc.subcore_barrier()
    base = (c * NS + s) * ROWS_PER_W
    pltpu.sync_copy(src_hbm.at[pl.ds(base, ROWS_PER_W)], src_v)
    pltpu.sync_copy(dst_hbm.at[pl.ds(base, ROWS_PER_W)], dst_v)

    def chunk(j, carry):
        pltpu.async_copy(table_hbm.at[src_v.at[j]], msg_v, sem).wait()
        pltpu.sync_copy(msg_v, agg_sh.at[dst_v.at[j]], add=True)
        return carry

    lax.fori_loop(0, ROWS_PER_W, chunk, 0)
    plsc.subcore_barrier()
    pltpu.sync_copy(agg_sh.at[pl.ds(s * NPS, NPS)],
                    out_hbm.at[c, pl.ds(s * NPS, NPS)])


@functools.partial(
    pl.kernel,
    out_type=jax.ShapeDtypeStruct((NC, NP, D_IN), jnp.float32),
    mesh=_mesh,
    scratch_types=[
        pltpu.VMEM((ROWS_PER_W, CHUNK), jnp.int32),
        pltpu.VMEM((ROWS_PER_W, CHUNK), jnp.int32),
        pltpu.VMEM((CHUNK, D_IN), jnp.float32),
        pltpu.VMEM((ZR, D_IN), jnp.float32),
        pltpu.VMEM_SHARED((NP, D_IN), jnp.float32),
        pltpu.SemaphoreType.DMA,
    ],
)
def _agg2_kernel(taba_hbm, tabb_hbm, src_hbm, dst_hbm, out_hbm,
                 src_v, dst_v, msg_v, zbuf, agg_sh, sem):
    c = lax.axis_index("c")
    s = lax.axis_index("s")
    _fill_const(zbuf, ZR, D_IN, 0.0)
    for k in range(NPS // ZR):
        pltpu.sync_copy(zbuf, agg_sh.at[pl.ds(s * NPS + k * ZR, ZR)])
    plsc.subcore_barrier()

    def run(table):
        for p in range(ROWS_PER_S // ROWS_PER_W):
            base = s * ROWS_PER_S + p * ROWS_PER_W
            pltpu.sync_copy(src_hbm.at[pl.ds(base, ROWS_PER_W)], src_v)
            pltpu.sync_copy(dst_hbm.at[pl.ds(base, ROWS_PER_W)], dst_v)

            def chunk(j, carry):
                pltpu.async_copy(table.at[src_v.at[j]], msg_v, sem).wait()
                pltpu.sync_copy(msg_v, agg_sh.at[dst_v.at[j]], add=True)
                return carry

            lax.fori_loop(0, ROWS_PER_W, chunk, 0)

    @pl.when(c == 0)
    def _():
        run(taba_hbm)

    @pl.when(c == 1)
    def _():
        run(tabb_hbm)

    plsc.subcore_barrier()
    pltpu.sync_copy(agg_sh.at[pl.ds(s * NPS, NPS)],
                    out_hbm.at[c, pl.ds(s * NPS, NPS)])


def _prep_body(x_ref, ds_ref, dd_ref, hs1_ref, ns_ref, nd_ref):
    ns = lax.rsqrt(jnp.maximum(ds_ref[:, :1], 1.0))
    nd = lax.rsqrt(jnp.maximum(dd_ref[:, :1], 1.0))
    hs1_ref[...] = x_ref[...] * ns
    ns_ref[...] = jnp.broadcast_to(ns, (BN, D_IN))
    nd_ref[...] = jnp.broadcast_to(nd, (BN, D_IN))


def _prep_call(x, degs, degd):
    return pl.pallas_call(
        _prep_body,
        grid=(NP // BN,),
        in_specs=[
            pl.BlockSpec((BN, D_IN), lambda i: (i, 0)),
            pl.BlockSpec((BN, 16), lambda i: (i, 0)),
            pl.BlockSpec((BN, 16), lambda i: (i, 0)),
        ],
        out_specs=[
            pl.BlockSpec((BN, D_IN), lambda i: (i, 0)),
            pl.BlockSpec((BN, D_IN), lambda i: (i, 0)),
            pl.BlockSpec((BN, D_IN), lambda i: (i, 0)),
        ],
        out_shape=[
            jax.ShapeDtypeStruct((NP, D_IN), jnp.float32),
            jax.ShapeDtypeStruct((NP, D_IN), jnp.float32),
            jax.ShapeDtypeStruct((NP, D_IN), jnp.float32),
        ],
    )(x, degs, degd)


def _l1_body(a0_ref, a1_ref, nd_ref, ns_ref, w_ref, b_ref, outa_ref, outb_ref):
    agg = (a0_ref[...] + a1_ref[...]) * nd_ref[...]
    h = jnp.dot(agg, w_ref[...], preferred_element_type=jnp.float32,
                precision=lax.Precision.HIGHEST) + b_ref[...]
    h = jnp.maximum(h, 0.0) * ns_ref[:, :1]
    outa_ref[...] = h[:, :D_IN]
    outb_ref[...] = h[:, D_IN:]


def _l1_call(a0, a1, nd, ns, W1, b1):
    return pl.pallas_call(
        _l1_body,
        grid=(NP // BN,),
        in_specs=[
            pl.BlockSpec((BN, D_IN), lambda i: (i, 0)),
            pl.BlockSpec((BN, D_IN), lambda i: (i, 0)),
            pl.BlockSpec((BN, D_IN), lambda i: (i, 0)),
            pl.BlockSpec((BN, D_IN), lambda i: (i, 0)),
            pl.BlockSpec((D_IN, H), lambda i: (0, 0)),
            pl.BlockSpec((1, H), lambda i: (0, 0)),
        ],
        out_specs=[
            pl.BlockSpec((BN, D_IN), lambda i: (i, 0)),
            pl.BlockSpec((BN, D_IN), lambda i: (i, 0)),
        ],
        out_shape=[
            jax.ShapeDtypeStruct((NP, D_IN), jnp.float32),
            jax.ShapeDtypeStruct((NP, D_IN), jnp.float32),
        ],
    )(a0, a1, nd, ns, W1, b1)


def _l2_body(aa_ref, ab_ref, nd_ref, w2a_ref, w2b_ref, b2_ref,
             wc1_ref, bc1_ref, wc3_ref, bc3_ref, out_ref, acc_ref):
    i = pl.program_id(0)

    @pl.when(i == 0)
    def _():
        acc_ref[...] = jnp.zeros_like(acc_ref)

    a = aa_ref[...] * nd_ref[...]
    b = ab_ref[...] * nd_ref[...]
    h = (jnp.dot(a, w2a_ref[...], preferred_element_type=jnp.float32,
                 precision=lax.Precision.HIGHEST)
         + jnp.dot(b, w2b_ref[...], preferred_element_type=jnp.float32,
                   precision=lax.Precision.HIGHEST)
         + b2_ref[...])
    h = jnp.maximum(h, 0.0)
    row = i * BN + lax.broadcasted_iota(jnp.int32, (BN, 1), 0)
    h = jnp.where(row < N, h, 0.0)
    acc_ref[...] += jnp.sum(h, axis=0, keepdims=True)

    @pl.when(i == pl.num_programs(0) - 1)
    def _():
        hg = acc_ref[...] * (1.0 / N)
        z = jnp.maximum(jnp.dot(hg, wc1_ref[...],
                                preferred_element_type=jnp.float32,
                                precision=lax.Precision.HIGHEST)
                        + bc1_ref[...], 0.0)
        out_ref[...] = jnp.dot(z, wc3_ref[...],
                               preferred_element_type=jnp.float32,
                               precision=lax.Precision.HIGHEST) + bc3_ref[...]


def _l2_call(aa, ab, nd, W2a, W2b, b2, Wc1, bc1, Wc3, bc3):
    return pl.pallas_call(
        _l2_body,
        grid=(NP // BN,),
        in_specs=[
            pl.BlockSpec((BN, D_IN), lambda i: (i, 0)),
            pl.BlockSpec((BN, D_IN), lambda i: (i, 0)),
            pl.BlockSpec((BN, D_IN), lambda i: (i, 0)),
            pl.BlockSpec((D_IN, H), lambda i: (0, 0)),
            pl.BlockSpec((D_IN, H), lambda i: (0, 0)),
            pl.BlockSpec((1, H), lambda i: (0, 0)),
            pl.BlockSpec((H, H), lambda i: (0, 0)),
            pl.BlockSpec((1, H), lambda i: (0, 0)),
            pl.BlockSpec((H, 1), lambda i: (0, 0)),
            pl.BlockSpec((1, 1), lambda i: (0, 0)),
        ],
        out_specs=pl.BlockSpec((1, 1), lambda i: (0, 0)),
        out_shape=jax.ShapeDtypeStruct((1, 1), jnp.float32),
        scratch_shapes=[pltpu.VMEM((1, H), jnp.float32)],
    )(aa, ab, nd, W2a, W2b, b2, Wc1, bc1, Wc3, bc3)


def kernel(x, edge_index, W1, b1, W2, b2, Wc1, bc1, Wc3, bc3):
    src2 = edge_index[0].reshape(EROWS, CHUNK)
    dst2 = edge_index[1].reshape(EROWS, CHUNK)
    xp = jnp.pad(x, ((0, NP - N), (0, 0)))

    degs = _deg_kernel(src2, dst2)
    hs1, ns, nd = _prep_call(xp, degs[0], degs[1])
    agg1 = _agg1_kernel(hs1, src2, dst2)
    h1a, h1b = _l1_call(agg1[0], agg1[1], nd, ns, W1, b1.reshape(1, H))
    agg2 = _agg2_kernel(h1a, h1b, src2, dst2)
    out = _l2_call(agg2[0], agg2[1], nd,
                   W2[:D_IN], W2[D_IN:], b2.reshape(1, H),
                   Wc1, bc1.reshape(1, H), Wc3, bc3.reshape(1, 1))
    return out
